# baseline (speedup 1.0000x reference)
import functools

import jax
import jax.numpy as jnp
from jax import lax
from jax.experimental import pallas as pl
from jax.experimental.pallas import tpu as pltpu
from jax.experimental.pallas import tpu_sc as plsc

_INFO = plsc.get_sparse_core_info()
_NC = _INFO.num_cores
_NW = _INFO.num_cores * _INFO.num_subcores

_V = 1000000
_D = 32
_B = 819200
_SEQ = 200
_BATCH = 4096

_FULL_TILES = _V // 128
_TAIL = _V - _FULL_TILES * 128
_CHUNK = 512
_UNITS = _B // _CHUNK

_MESH = plsc.VectorSubcoreMesh(core_axis_name="c", subcore_axis_name="s")


def _wid():
    return lax.axis_index("s") * _NC + lax.axis_index("c")


@functools.partial(
    pl.kernel,
    mesh=_MESH,
    out_type=jax.ShapeDtypeStruct((_V // 4, 128), jnp.float32),
    scratch_types=[
        pltpu.VMEM((32, 128), jnp.float32),
        pltpu.VMEM((32, 128), jnp.float32),
    ],
    compiler_params=pltpu.CompilerParams(
        use_tc_tiling_on_sc=True, needs_layout_passes=False
    ),
)
def _convert(tableT_hbm, tail_hbm, scratch_hbm, src_v, dst_v):
    w = _wid()
    io16 = lax.iota(jnp.int32, 16)

    def transpose_tile(n_r):
        for r in range(n_r):
            for k in range(8):
                rvec = io16 + (16 * (k & 1))
                cvec = jnp.full((16,), 4 * r + (k >> 1), jnp.int32)
                v = plsc.load_gather(src_v, [rvec, cvec])
                dst_v[r, pl.ds(16 * k, 16)] = v

    n_tiles = 244 + jnp.where(w < 4, 1, 0).astype(jnp.int32)

    def body(i, carry):
        tc = i * _NW + w
        pltpu.sync_copy(tableT_hbm.at[:, pl.ds(tc * 128, 128)], src_v)
        transpose_tile(32)
        pltpu.sync_copy(dst_v, scratch_hbm.at[pl.ds(tc * 32, 32)])
        return carry

    lax.fori_loop(0, n_tiles, body, 0)

    @pl.when(w == 4)
    def _():
        pltpu.sync_copy(tail_hbm, src_v.at[pl.ds(0, _TAIL // 4)])
        pltpu.sync_copy(
            src_v.at[pl.ds(0, _TAIL // 4)],
            scratch_hbm.at[pl.ds(_FULL_TILES * 32, _TAIL // 4)],
        )


@functools.partial(
    pl.kernel,
    mesh=_MESH,
    out_type=jax.ShapeDtypeStruct((_SEQ, 4, 32, 8, 128), jnp.float32),
    scratch_types=[
        pltpu.VMEM((_CHUNK,), jnp.int32),
        pltpu.VMEM((_CHUNK, _D), jnp.float32),
        pltpu.VMEM((4, 4, 8, 128), jnp.float32),
        pltpu.SemaphoreType.DMA,
    ],
    compiler_params=pltpu.CompilerParams(
        use_tc_tiling_on_sc=False, needs_layout_passes=False
    ),
)
def _gather(xf_hbm, scratch_hbm, out5_hbm, idx_v, rows_v, fmt_v, gsem):
    w = _wid()
    io16 = lax.iota(jnp.int32, 16)

    def body(u, carry):
        unit = u * _NW + w
        s = unit // 8
        tcg = unit % 8
        pltpu.sync_copy(xf_hbm.at[pl.ds(unit * _CHUNK, _CHUNK)], idx_v)
        pltpu.async_copy(scratch_hbm.at[idx_v], rows_v, gsem).wait()
        for tcl in range(4):
            for k in range(8):
                bvec = io16 + (128 * tcl + 16 * k)
                for tr in range(4):
                    for r in range(8):
                        dvec = jnp.full((16,), 8 * tr + r, jnp.int32)
                        v = plsc.load_gather(rows_v, [bvec, dvec])
                        fmt_v[tr, tcl, r, pl.ds(16 * k, 16)] = v
        for tr in range(4):
            pltpu.sync_copy(
                fmt_v.at[tr], out5_hbm.at[s, tr, pl.ds(tcg * 4, 4)]
            )
        return carry

    lax.fori_loop(0, _UNITS // _NW, body, 0)


def kernel(x, table):
    tableT = table.T
    tail = table[_FULL_TILES * 128 :].reshape(_TAIL // 4, 128)
    scratch = _convert(tableT, tail)
    xf = x.T.reshape(_B)
    scratch2 = scratch.reshape(_V, _D)
    out5 = _gather(xf, scratch2)
    return out5.transpose((2, 4, 0, 1, 3)).reshape(_BATCH, _SEQ, _D)


# speedup vs baseline: 1.5138x; 1.5138x over previous
import functools

import jax
import jax.numpy as jnp
from jax import lax
from jax.experimental import pallas as pl
from jax.experimental.pallas import tpu as pltpu
from jax.experimental.pallas import tpu_sc as plsc

_INFO = plsc.get_sparse_core_info()
_NC = _INFO.num_cores
_NW = _INFO.num_cores * _INFO.num_subcores

_V = 1000000
_D = 32
_B = 819200
_SEQ = 200
_BATCH = 4096

_FULL_TILES = _V // 128
_TAIL = _V - _FULL_TILES * 128
_CHUNK = 512
_UNITS = _B // _CHUNK

_MESH = plsc.VectorSubcoreMesh(core_axis_name="c", subcore_axis_name="s")


def _wid():
    return lax.axis_index("s") * _NC + lax.axis_index("c")


@functools.partial(
    pl.kernel,
    mesh=_MESH,
    out_type=jax.ShapeDtypeStruct((_V // 4, 128), jnp.float32),
    scratch_types=[
        pltpu.VMEM((32, 128), jnp.float32),
        pltpu.VMEM((32, 128), jnp.float32),
    ],
    compiler_params=pltpu.CompilerParams(
        use_tc_tiling_on_sc=True, needs_layout_passes=False
    ),
)
def _convert(tableT_hbm, tail_hbm, scratch_hbm, src_v, dst_v):
    w = _wid()
    io16 = lax.iota(jnp.int32, 16)

    def transpose_tile(n_r):
        @plsc.parallel_loop(0, n_r * 8, unroll=8)
        def _(i):
            r = i >> 3
            k = i & 7
            rvec = io16 + ((k & 1) << 4)
            cvec = jnp.full((16,), 0, jnp.int32) + (4 * r + (k >> 1))
            v = plsc.load_gather(src_v, [rvec, cvec])
            dst_v[r, pl.ds(16 * (i & 7), 16)] = v

    n_tiles = 244 + jnp.where(w < 4, 1, 0).astype(jnp.int32)

    def body(i, carry):
        tc = i * _NW + w
        pltpu.sync_copy(tableT_hbm.at[:, pl.ds(tc * 128, 128)], src_v)
        transpose_tile(32)
        pltpu.sync_copy(dst_v, scratch_hbm.at[pl.ds(tc * 32, 32)])
        return carry

    lax.fori_loop(0, n_tiles, body, 0)

    @pl.when(w == 4)
    def _():
        pltpu.sync_copy(tail_hbm, src_v.at[pl.ds(0, _TAIL // 4)])
        pltpu.sync_copy(
            src_v.at[pl.ds(0, _TAIL // 4)],
            scratch_hbm.at[pl.ds(_FULL_TILES * 32, _TAIL // 4)],
        )


@functools.partial(
    pl.kernel,
    mesh=_MESH,
    out_type=jax.ShapeDtypeStruct((_SEQ, 4, 32, 8, 128), jnp.float32),
    scratch_types=[
        pltpu.VMEM((_CHUNK,), jnp.int32),
        pltpu.VMEM((_CHUNK, _D), jnp.float32),
        pltpu.VMEM((4, 4, 8, 128), jnp.float32),
        pltpu.SemaphoreType.DMA,
    ],
    compiler_params=pltpu.CompilerParams(
        use_tc_tiling_on_sc=False, needs_layout_passes=False
    ),
)
def _gather(xf_hbm, scratch_hbm, out5_hbm, idx_v, rows_v, fmt_v, gsem):
    w = _wid()
    io16 = lax.iota(jnp.int32, 16)

    def body(u, carry):
        unit = u * _NW + w
        s = unit // 8
        tcg = unit % 8
        pltpu.sync_copy(xf_hbm.at[pl.ds(unit * _CHUNK, _CHUNK)], idx_v)
        pltpu.async_copy(scratch_hbm.at[idx_v], rows_v, gsem).wait()
        @plsc.parallel_loop(0, 1024, unroll=8)
        def _(i):
            tcl = i >> 8
            k = (i >> 5) & 7
            tr = (i >> 3) & 3
            r = i & 7
            bvec = io16 + ((tcl << 7) + (k << 4))
            dvec = jnp.full((16,), 0, jnp.int32) + ((tr << 3) + r)
            v = plsc.load_gather(rows_v, [bvec, dvec])
            fmt_v[tr, tcl, r, pl.ds(k << 4, 16)] = v
        for tr in range(4):
            pltpu.sync_copy(
                fmt_v.at[tr], out5_hbm.at[s, tr, pl.ds(tcg * 4, 4)]
            )
        return carry

    lax.fori_loop(0, _UNITS // _NW, body, 0)


def kernel(x, table):
    tableT = table.T
    tail = table[_FULL_TILES * 128 :].reshape(_TAIL // 4, 128)
    scratch = _convert(tableT, tail)
    xf = x.T.reshape(_B)
    scratch2 = scratch.reshape(_V, _D)
    out5 = _gather(xf, scratch2)
    return out5.transpose((2, 4, 0, 1, 3)).reshape(_BATCH, _SEQ, _D)


# speedup vs baseline: 1.9774x; 1.3063x over previous
import functools

import jax
import jax.numpy as jnp
from jax import lax
from jax.experimental import pallas as pl
from jax.experimental.pallas import tpu as pltpu
from jax.experimental.pallas import tpu_sc as plsc

_INFO = plsc.get_sparse_core_info()
_NC = _INFO.num_cores
_NW = _INFO.num_cores * _INFO.num_subcores

_V = 1000000
_D = 32
_B = 819200
_SEQ = 200
_BATCH = 4096

_CHUNK = 512
_UNITS = _B // _CHUNK
_UPW = _UNITS // _NW

_MESH = plsc.VectorSubcoreMesh(core_axis_name="c", subcore_axis_name="s")


@functools.partial(
    pl.kernel,
    mesh=_MESH,
    out_type=jax.ShapeDtypeStruct((_SEQ, 4, 32, 8, 128), jnp.float32),
    scratch_types=[
        [pltpu.VMEM((_CHUNK,), jnp.int32) for _ in range(2)],
        [pltpu.VMEM((_CHUNK, _D), jnp.float32) for _ in range(2)],
        [pltpu.VMEM((4, 4, 8, 128), jnp.float32) for _ in range(2)],
        [pltpu.SemaphoreType.DMA for _ in range(2)],
        [pltpu.SemaphoreType.DMA for _ in range(2)],
    ],
    compiler_params=pltpu.CompilerParams(
        use_tc_tiling_on_sc=False, needs_layout_passes=False
    ),
)
def _gather(xf_hbm, table_hbm, out5_hbm, idx, rows, fmt, gsem, wsem):
    w = lax.axis_index("s") * _NC + lax.axis_index("c")
    io16 = lax.iota(jnp.int32, 16)
    dvecs = [jnp.full((16,), d, jnp.int32) for d in range(_D)]

    def load_unit(u, b):
        pltpu.sync_copy(xf_hbm.at[pl.ds((u * _NW + w) * _CHUNK, _CHUNK)], idx[b])
        pltpu.async_copy(table_hbm.at[idx[b]], rows[b], gsem[b])

    def wait_gather(b):
        pltpu.make_async_copy(table_hbm.at[pl.ds(0, _CHUNK)], rows[b], gsem[b]).wait()

    def wait_writes(b):
        for tr in range(4):
            pltpu.make_async_copy(
                fmt[b].at[tr], out5_hbm.at[0, 0, pl.ds(0, 4)], wsem[b]
            ).wait()

    def transpose_unit(b):
        @plsc.parallel_loop(0, 32, unroll=2)
        def _(i):
            tcl = i >> 3
            k = i & 7
            bvec = io16 + ((tcl << 7) + (k << 4))
            for tr in range(4):
                for r in range(8):
                    v = plsc.load_gather(rows[b], [bvec, dvecs[8 * tr + r]])
                    fmt[b][tr, tcl, r, pl.ds(k << 4, 16)] = v

    def write_unit(u, b):
        unit = u * _NW + w
        s = unit // 8
        tcg = unit % 8
        for tr in range(4):
            pltpu.async_copy(
                fmt[b].at[tr], out5_hbm.at[s, tr, pl.ds(tcg * 4, 4)], wsem[b]
            )

    load_unit(0, 0)

    def body(g, carry):
        for sb in range(2):
            u = g * 2 + sb

            @pl.when(u + 1 < _UPW)
            def _(nb=sb ^ 1, u=u):
                load_unit(u + 1, nb)

            wait_gather(sb)

            @pl.when(u >= 2)
            def _(sb=sb):
                wait_writes(sb)

            transpose_unit(sb)
            write_unit(u, sb)
        return carry

    lax.fori_loop(0, _UPW // 2, body, 0)
    wait_writes(0)
    wait_writes(1)


def kernel(x, table):
    t4 = lax.optimization_barrier(table.reshape(_V // 4, 128))
    t_lin = t4.reshape(_V, _D)
    xf = x.T.reshape(_B)
    out5 = _gather(xf, t_lin)
    return out5.transpose((2, 4, 0, 1, 3)).reshape(_BATCH, _SEQ, _D)


# speedup vs baseline: 2.0481x; 1.0357x over previous
import functools

import jax
import jax.numpy as jnp
from jax import lax
from jax.experimental import pallas as pl
from jax.experimental.pallas import tpu as pltpu
from jax.experimental.pallas import tpu_sc as plsc

_INFO = plsc.get_sparse_core_info()
_NC = _INFO.num_cores
_NW = _INFO.num_cores * _INFO.num_subcores

_V = 1000000
_D = 32
_B = 819200
_SEQ = 200
_BATCH = 4096

_CHUNK = 512
_UNITS = _B // _CHUNK
_UPW = _UNITS // _NW

_MESH = plsc.VectorSubcoreMesh(core_axis_name="c", subcore_axis_name="s")


@functools.partial(
    pl.kernel,
    mesh=_MESH,
    out_type=jax.ShapeDtypeStruct((_SEQ, 4, 32, 8, 128), jnp.float32),
    scratch_types=[
        [pltpu.VMEM((_CHUNK,), jnp.int32) for _ in range(2)],
        [pltpu.VMEM((_CHUNK, _D), jnp.float32) for _ in range(2)],
        [pltpu.VMEM((4, 4, 8, 128), jnp.float32) for _ in range(2)],
        [pltpu.SemaphoreType.DMA for _ in range(2)],
        [pltpu.SemaphoreType.DMA for _ in range(2)],
    ],
    compiler_params=pltpu.CompilerParams(
        use_tc_tiling_on_sc=False, needs_layout_passes=False
    ),
)
def _gather(xf_hbm, table_hbm, out5_hbm, idx, rows, fmt, gsem, wsem):
    w = lax.axis_index("s") * _NC + lax.axis_index("c")
    io16 = lax.iota(jnp.int32, 16)
    dvecs = [jnp.full((16,), d, jnp.int32) for d in range(_D)]

    def load_unit(u, b):
        pltpu.sync_copy(xf_hbm.at[pl.ds((u * _NW + w) * _CHUNK, _CHUNK)], idx[b])
        pltpu.async_copy(table_hbm.at[idx[b]], rows[b], gsem[b])

    def wait_gather(b):
        pltpu.make_async_copy(table_hbm.at[pl.ds(0, _CHUNK)], rows[b], gsem[b]).wait()

    def wait_writes(b):
        for tr in range(4):
            pltpu.make_async_copy(
                fmt[b].at[tr], out5_hbm.at[0, 0, pl.ds(0, 4)], wsem[b]
            ).wait()

    def transpose_unit(b):
        @plsc.parallel_loop(0, 32, unroll=2)
        def _(i):
            tcl = i >> 3
            k = i & 7
            bvec = io16 + ((tcl << 7) + (k << 4))
            for tr in range(4):
                vs = [
                    plsc.load_gather(rows[b], [bvec, dvecs[8 * tr + r]])
                    for r in range(8)
                ]
                for r in range(8):
                    fmt[b][tr, tcl, r, pl.ds(k << 4, 16)] = vs[r]

    def write_unit(u, b):
        unit = u * _NW + w
        s = unit // 8
        tcg = unit % 8
        for tr in range(4):
            pltpu.async_copy(
                fmt[b].at[tr], out5_hbm.at[s, tr, pl.ds(tcg * 4, 4)], wsem[b]
            )

    load_unit(0, 0)

    def body(g, carry):
        for sb in range(2):
            u = g * 2 + sb

            @pl.when(u + 1 < _UPW)
            def _(nb=sb ^ 1, u=u):
                load_unit(u + 1, nb)

            wait_gather(sb)

            @pl.when(u >= 2)
            def _(sb=sb):
                wait_writes(sb)

            transpose_unit(sb)
            write_unit(u, sb)
        return carry

    lax.fori_loop(0, _UPW // 2, body, 0)
    wait_writes(0)
    wait_writes(1)


def kernel(x, table):
    t4 = lax.optimization_barrier(table.reshape(_V // 4, 128))
    t_lin = t4.reshape(_V, _D)
    xf = x.T.reshape(_B)
    out5 = _gather(xf, t_lin)
    return out5.transpose((2, 4, 0, 1, 3)).reshape(_BATCH, _SEQ, _D)
